# baseline (speedup 1.0000x reference)
import functools

import jax
import jax.numpy as jnp
from jax.experimental import pallas as pl
from jax.experimental.pallas import tpu as pltpu


def _fused_kernel(nk, x_ref, A_ref, W_ref, out_ref,
                  acc_ref, deg_ref, abf_ref, z2_ref, w1_ref, c1_ref):
    l = pl.program_id(0)
    k = pl.program_id(1)
    bk = abf_ref.shape[1]
    D = out_ref.shape[1]

    @pl.when(jnp.logical_and(l == 0, k == 0))
    def _prologue():
        acc_ref[...] = jnp.zeros_like(acc_ref)
        deg_ref[...] = jnp.zeros_like(deg_ref)
        U0 = W_ref[0:D, :]
        U1 = W_ref[D:2 * D, :]
        b0r = W_ref[3 * D:3 * D + 1, :]
        w1_ref[...] = jax.lax.dot_general(
            U1, U0, (((1,), (1,)), ((), ())),
            preferred_element_type=jnp.float32).astype(jnp.bfloat16)
        c1_ref[...] = jax.lax.dot_general(
            U1, b0r, (((1,), (1,)), ((), ())),
            preferred_element_type=jnp.float32)

    @pl.when(l == 0)
    def _layer1_step():
        A_raw = A_ref[...]
        Af = A_raw.astype(jnp.bfloat16)
        abf_ref[k] = Af
        z1 = jax.lax.dot_general(
            w1_ref[...], x_ref[...].astype(jnp.bfloat16),
            (((1,), (1,)), ((), ())),
            preferred_element_type=jnp.float32) + c1_ref[...]
        z1a = jnp.concatenate(
            [z1.astype(jnp.bfloat16),
             jnp.ones((8, z1.shape[1]), jnp.bfloat16)], axis=0)
        acc_ref[...] += jnp.dot(z1a, Af, preferred_element_type=jnp.float32)

        @pl.when(k == nk - 1)
        def _layer1_out():
            inv = 1.0 / acc_ref[D:D + 1, :]
            deg_ref[...] = inv
            y1 = jnp.maximum(acc_ref[0:D, :] * inv, 0.0)
            U2 = W_ref[2 * D:3 * D, :]
            z2_ref[...] = jnp.dot(U2.astype(jnp.bfloat16),
                                  y1.astype(jnp.bfloat16),
                                  preferred_element_type=jnp.float32
                                  ).astype(jnp.bfloat16)

    @pl.when(l == 1)
    def _layer2_step():
        acc2 = jnp.zeros((D, bk), jnp.float32)
        for j in range(nk):
            acc2 += jnp.dot(z2_ref[:, j * bk:(j + 1) * bk],
                            abf_ref[j, :, pl.ds(k * bk, bk)],
                            preferred_element_type=jnp.float32)
        aggT = jnp.maximum(acc2 * deg_ref[:, pl.ds(k * bk, bk)], 0.0)
        out_ref[...] = aggT.T


def kernel(x, adj_mat, U0, b0, U1, U2):
    N, D = x.shape
    bk = 512
    nk = N // bk
    W = jnp.concatenate([U0, U1, U2, b0.reshape(1, D),
                         jnp.zeros((7, D), jnp.float32)], axis=0)
    return pl.pallas_call(
        functools.partial(_fused_kernel, nk),
        grid=(2, nk),
        in_specs=[
            pl.BlockSpec((bk, D),
                         lambda l, k: (jnp.where(l == 0, k, nk - 1), 0)),
            pl.BlockSpec((bk, N),
                         lambda l, k: (jnp.where(l == 0, k, nk - 1), 0)),
            pl.BlockSpec((3 * D + 8, D), lambda l, k: (0, 0)),
        ],
        out_specs=pl.BlockSpec((bk, D),
                               lambda l, k: (jnp.where(l == 0, 0, k), 0)),
        out_shape=jax.ShapeDtypeStruct((N, D), jnp.float32),
        scratch_shapes=[
            pltpu.VMEM((D + 8, N), jnp.float32),
            pltpu.VMEM((1, N), jnp.float32),
            pltpu.VMEM((nk, bk, N), jnp.bfloat16),
            pltpu.VMEM((D, N), jnp.bfloat16),
            pltpu.VMEM((D, D), jnp.bfloat16),
            pltpu.VMEM((D, 1), jnp.float32),
        ],
        compiler_params=pltpu.CompilerParams(
            dimension_semantics=("arbitrary", "arbitrary")),
    )(x, adj_mat, W)


# speedup vs baseline: 1.1405x; 1.1405x over previous
import functools

import jax
import jax.numpy as jnp
from jax.experimental import pallas as pl
from jax.experimental.pallas import tpu as pltpu


def _fused_kernel(nk, x_ref, A_ref, U0_ref, b0_ref, U1_ref, U2_ref, out_ref,
                  acc_ref, deg_ref, abf_ref, z2_ref, w1_ref, c1_ref):
    l = pl.program_id(0)
    k = pl.program_id(1)
    bk = out_ref.shape[0]
    D = acc_ref.shape[0]

    @pl.when(jnp.logical_and(l == 0, k == 0))
    def _prologue():
        acc_ref[...] = jnp.zeros_like(acc_ref)
        deg_ref[...] = jnp.zeros_like(deg_ref)
        w1_ref[...] = jax.lax.dot_general(
            U1_ref[...], U0_ref[...], (((1,), (1,)), ((), ())),
            preferred_element_type=jnp.float32).astype(jnp.bfloat16)
        c1_ref[...] = jnp.dot(U1_ref[...], b0_ref[...],
                              preferred_element_type=jnp.float32)

    @pl.when(l == 0)
    def _layer1_step():
        A_raw = A_ref[...]
        Af = A_raw.astype(jnp.bfloat16)
        abf_ref[pl.ds(k * bk, bk), :] = Af
        z1 = jax.lax.dot_general(
            w1_ref[...], x_ref[...].astype(jnp.bfloat16),
            (((1,), (1,)), ((), ())),
            preferred_element_type=jnp.float32) + c1_ref[...]
        acc_ref[...] += jnp.dot(z1.astype(jnp.bfloat16), Af,
                                preferred_element_type=jnp.float32)
        deg_ref[...] += jnp.sum(A_raw, axis=0, keepdims=True).astype(jnp.float32)

        @pl.when(k == nk - 1)
        def _layer1_out():
            inv = 1.0 / deg_ref[...]
            deg_ref[...] = inv
            y1 = jnp.maximum(acc_ref[...] * inv, 0.0)
            z2_ref[...] = jnp.dot(U2_ref[...].astype(jnp.bfloat16),
                                  y1.astype(jnp.bfloat16),
                                  preferred_element_type=jnp.float32
                                  ).astype(jnp.bfloat16)

    @pl.when(l == 1)
    def _layer2_step():
        acc2 = jnp.dot(z2_ref[...], abf_ref[:, pl.ds(k * bk, bk)],
                       preferred_element_type=jnp.float32)
        aggT = jnp.maximum(acc2 * deg_ref[:, pl.ds(k * bk, bk)], 0.0)
        out_ref[...] = aggT.T


def kernel(x, adj_mat, U0, b0, U1, U2):
    N, D = x.shape
    bk = 512
    nk = N // bk
    b0c = b0.reshape(D, 1)
    return pl.pallas_call(
        functools.partial(_fused_kernel, nk),
        grid=(2, nk),
        in_specs=[
            pl.BlockSpec((bk, D),
                         lambda l, k: (jnp.where(l == 0, k, nk - 1), 0)),
            pl.BlockSpec((bk, N),
                         lambda l, k: (jnp.where(l == 0, k, nk - 1), 0)),
            pl.BlockSpec((D, D), lambda l, k: (0, 0)),
            pl.BlockSpec((D, 1), lambda l, k: (0, 0)),
            pl.BlockSpec((D, D), lambda l, k: (0, 0)),
            pl.BlockSpec((D, D), lambda l, k: (0, 0)),
        ],
        out_specs=pl.BlockSpec((bk, D),
                               lambda l, k: (jnp.where(l == 0, 0, k), 0)),
        out_shape=jax.ShapeDtypeStruct((N, D), jnp.float32),
        scratch_shapes=[
            pltpu.VMEM((D, N), jnp.float32),
            pltpu.VMEM((1, N), jnp.float32),
            pltpu.VMEM((N, N), jnp.bfloat16),
            pltpu.VMEM((D, N), jnp.bfloat16),
            pltpu.VMEM((D, D), jnp.bfloat16),
            pltpu.VMEM((D, 1), jnp.float32),
        ],
        compiler_params=pltpu.CompilerParams(
            dimension_semantics=("arbitrary", "arbitrary")),
    )(x, adj_mat, U0, b0c, U1, U2)
